# baseline (speedup 1.0000x reference)
import jax
import jax.numpy as jnp
from jax import lax
from jax.experimental import pallas as pl
from jax.experimental.pallas import tpu as pltpu
from jax.experimental.pallas import tpu_sc as plsc

N = 6_400_000
S = 100_000
NWORK = 32
PERW = N // NWORK
C = 20_000
NCH = PERW // C
A = 32_768
FB = 4_096
NFB = A // FB
OUT_PAD = 100_352
ACC_PAD = 133_120
ZSL = ACC_PAD // 16


def _sc_body(ind_hbm, val_hbm, out_hbm, ids0, ids1, vls0, vls1, win,
             elemidx, acc, si0, si1, sv0, sv1):
    c = lax.axis_index("c")
    s = lax.axis_index("s")
    w = s * 2 + c
    base = w * PERW
    z16 = jnp.zeros((16,), jnp.float32)
    iota16 = lax.iota(jnp.int32, 16)
    lane15 = iota16 == 15

    def flush(W, ph):
        used = ph - W + 1
        for b in range(NFB):
            @pl.when(b * FB < used)
            def _():
                def fe(j, carry):
                    elemidx[pl.ds(j * 16, 16)] = W + b * FB + j * 16 + iota16
                    return carry
                lax.fori_loop(0, FB // 16, fe, 0)
                pltpu.sync_copy(win.at[pl.ds(b * FB, FB)], acc.at[elemidx],
                                add=True)
                def zz(j, carry):
                    win[pl.ds(b * FB + j * 16, 16)] = z16
                    return carry
                lax.fori_loop(0, FB // 16, zz, 0)

    def align(x):
        return jnp.bitwise_and(x, jnp.int32(-128))

    def scatter(ids_ref, vls_ref, k, W):
        loc = ids_ref[pl.ds(k * 16, 16)] - W
        vv = vls_ref[pl.ds(k * 16, 16)]
        plsc.addupdate_scatter(win, [loc], vv)

    def process(ids_ref, vls_ref, Wp):
        W, ph = Wp
        lo = ids_ref[pl.ds(0, 16)][0]
        hi = ids_ref[pl.ds(C - 16, 16)][15]

        def chunk_flush(a):
            flush(a[0], a[1])
            return align(lo)
        W = lax.cond(hi >= W + A, chunk_flush, lambda a: a[0], (W, ph))

        def fastp(Wx):
            @plsc.parallel_loop(0, C // 16, unroll=8)
            def fb(k):
                b16 = k * 16
                ids = ids_ref[pl.ds(b16, 16)]
                vv = vls_ref[pl.ds(b16, 16)]
                idn = ids_ref[pl.ds(b16 + 1, 16)]
                prefix = plsc.cumsum(vv)
                bnd = ids != idn
                m_end = bnd | lane15
                plsc.addupdate_scatter(win, [ids - Wx], prefix, mask=m_end)
                m_sub = bnd & (~lane15)
                plsc.addupdate_scatter(win, [jnp.clip(idn - Wx, 0, A - 1)],
                                       -prefix, mask=m_sub)
            return Wx

        def slowp(Wx):
            def sb(k, Wy):
                ids = ids_ref[pl.ds(k * 16, 16)]
                vv = vls_ref[pl.ds(k * 16, 16)]
                def vec_flush(Wz):
                    flush(Wz, Wz + A - 1)
                    return align(ids[0])
                Wy = lax.cond(ids[15] >= Wy + A, vec_flush, lambda Wz: Wz, Wy)
                def one_pass(p, Wz):
                    loc = ids - Wz
                    m = (loc >= 0) & (loc < A)
                    plsc.addupdate_scatter(
                        win, [jnp.clip(loc, 0, A - 1)], vv, mask=m)
                    def adv(Wq):
                        flush(Wq, Wq + A - 1)
                        return Wq + A
                    return lax.cond(ids[15] >= Wz + A, adv,
                                    lambda Wq: Wq, Wz)
                return lax.fori_loop(0, 4, one_pass, Wy)
            return lax.fori_loop(0, C // 16, sb, Wx)

        W = lax.cond(hi < W + A, fastp, slowp, W)
        return (W, hi)

    def issue(buf_i, buf_v, sem_i, sem_v, chunk):
        off = base + chunk * C
        pltpu.make_async_copy(ind_hbm.at[pl.ds(off, C)],
                              buf_i.at[pl.ds(0, C)], sem_i).start()
        pltpu.make_async_copy(val_hbm.at[pl.ds(off, C)], buf_v, sem_v).start()

    def wait(buf_i, buf_v, sem_i, sem_v, chunk):
        off = base + chunk * C
        pltpu.make_async_copy(ind_hbm.at[pl.ds(off, C)],
                              buf_i.at[pl.ds(0, C)], sem_i).wait()
        pltpu.make_async_copy(val_hbm.at[pl.ds(off, C)], buf_v, sem_v).wait()

    def zw(j, carry):
        win[pl.ds(j * 16, 16)] = z16
        return carry
    lax.fori_loop(0, A // 16, zw, 0)
    ids0[pl.ds(C, 16)] = jnp.zeros((16,), jnp.int32)
    ids1[pl.ds(C, 16)] = jnp.zeros((16,), jnp.int32)
    pltpu.sync_copy(win.at[pl.ds(0, ZSL)], acc.at[pl.ds(s * ZSL, ZSL)])
    plsc.subcore_barrier()

    issue(ids0, vls0, si0, sv0, 0)

    def gbody(g, Wp):
        cA = 2 * g
        wait(ids0, vls0, si0, sv0, cA)
        issue(ids1, vls1, si1, sv1, cA + 1)
        Wp = process(ids0, vls0, Wp)
        wait(ids1, vls1, si1, sv1, cA + 1)
        @pl.when(g < NCH // 2 - 1)
        def _():
            issue(ids0, vls0, si0, sv0, cA + 2)
        Wp = process(ids1, vls1, Wp)
        return Wp

    W, ph = lax.fori_loop(0, NCH // 2, gbody,
                          (jnp.int32(0), jnp.int32(0)))
    flush(W, ph)

    plsc.subcore_barrier()

    @pl.when(s == 0)
    def _():
        pltpu.sync_copy(acc.at[pl.ds(0, OUT_PAD)], out_hbm.at[c])


def _merge_body(a_ref, o_ref):
    o_ref[...] = a_ref[0] + a_ref[1]


def kernel(ind_1, output):
    ind2 = ind_1.reshape(N)
    val2 = output.reshape(N)

    mesh = plsc.VectorSubcoreMesh(core_axis_name="c", subcore_axis_name="s")
    partials = pl.kernel(
        _sc_body,
        out_type=jax.ShapeDtypeStruct((2, OUT_PAD), jnp.float32),
        mesh=mesh,
        compiler_params=pltpu.CompilerParams(needs_layout_passes=False),
        scratch_types=[
            pltpu.VMEM((C + 16,), jnp.int32),
            pltpu.VMEM((C + 16,), jnp.int32),
            pltpu.VMEM((C,), jnp.float32),
            pltpu.VMEM((C,), jnp.float32),
            pltpu.VMEM((A,), jnp.float32),
            pltpu.VMEM((FB,), jnp.int32),
            pltpu.VMEM_SHARED((ACC_PAD,), jnp.float32),
            pltpu.SemaphoreType.DMA,
            pltpu.SemaphoreType.DMA,
            pltpu.SemaphoreType.DMA,
            pltpu.SemaphoreType.DMA,
        ],
    )(ind2, val2)

    merged = pl.pallas_call(
        _merge_body,
        out_shape=jax.ShapeDtypeStruct((OUT_PAD // 128, 128), jnp.float32),
    )(partials.reshape(2, OUT_PAD // 128, 128))
    return merged.reshape(OUT_PAD)[:S]


# speedup vs baseline: 1.0079x; 1.0079x over previous
import jax
import jax.numpy as jnp
from jax import lax
from jax.experimental import pallas as pl
from jax.experimental.pallas import tpu as pltpu
from jax.experimental.pallas import tpu_sc as plsc

N = 6_400_000
S = 100_000
NWORK = 32
PERW = N // NWORK
C = 10_000
NCH = PERW // C
A = 32_768
FB = 4_096
NFB = A // FB
OUT_PAD = 100_352
ACC_PAD = 133_120
ZSL = ACC_PAD // 16


def _sc_body(ind_hbm, val_hbm, out_hbm, ids0, ids1, vls0, vls1, win,
             elemidx, acc, si0, si1, sv0, sv1):
    c = lax.axis_index("c")
    s = lax.axis_index("s")
    w = s * 2 + c
    base = w * PERW
    z16 = jnp.zeros((16,), jnp.float32)
    iota16 = lax.iota(jnp.int32, 16)
    lane15 = iota16 == 15

    def flush(W, ph):
        used = ph - W + 1
        for b in range(NFB):
            @pl.when(b * FB < used)
            def _():
                def fe(j, carry):
                    elemidx[pl.ds(j * 16, 16)] = W + b * FB + j * 16 + iota16
                    return carry
                lax.fori_loop(0, FB // 16, fe, 0)
                pltpu.sync_copy(win.at[pl.ds(b * FB, FB)], acc.at[elemidx],
                                add=True)
                def zz(j, carry):
                    win[pl.ds(b * FB + j * 16, 16)] = z16
                    return carry
                lax.fori_loop(0, FB // 16, zz, 0)

    def align(x):
        return jnp.bitwise_and(x, jnp.int32(-128))

    def scatter(ids_ref, vls_ref, k, W):
        loc = ids_ref[pl.ds(k * 16, 16)] - W
        vv = vls_ref[pl.ds(k * 16, 16)]
        plsc.addupdate_scatter(win, [loc], vv)

    def process(ids_ref, vls_ref, Wp):
        W, ph = Wp
        lo = ids_ref[pl.ds(0, 16)][0]
        hi = ids_ref[pl.ds(C - 16, 16)][15]

        def chunk_flush(a):
            flush(a[0], a[1])
            return align(lo)
        W = lax.cond(hi >= W + A, chunk_flush, lambda a: a[0], (W, ph))

        def fastp(Wx):
            @plsc.parallel_loop(0, C // 16, unroll=4)
            def fb(k):
                b16 = k * 16
                ids = ids_ref[pl.ds(b16, 16)]
                vv = vls_ref[pl.ds(b16, 16)]
                idn = ids_ref[pl.ds(b16 + 1, 16)]
                prefix = plsc.cumsum(vv)
                bnd = ids != idn
                m_end = bnd | lane15
                plsc.addupdate_scatter(win, [ids - Wx], prefix, mask=m_end)
                m_sub = bnd & (~lane15)
                plsc.addupdate_scatter(win, [jnp.clip(idn - Wx, 0, A - 1)],
                                       -prefix, mask=m_sub)
            return Wx

        def slowp(Wx):
            def sb(k, Wy):
                ids = ids_ref[pl.ds(k * 16, 16)]
                vv = vls_ref[pl.ds(k * 16, 16)]
                def vec_flush(Wz):
                    flush(Wz, Wz + A - 1)
                    return align(ids[0])
                Wy = lax.cond(ids[15] >= Wy + A, vec_flush, lambda Wz: Wz, Wy)
                def one_pass(p, Wz):
                    loc = ids - Wz
                    m = (loc >= 0) & (loc < A)
                    plsc.addupdate_scatter(
                        win, [jnp.clip(loc, 0, A - 1)], vv, mask=m)
                    def adv(Wq):
                        flush(Wq, Wq + A - 1)
                        return Wq + A
                    return lax.cond(ids[15] >= Wz + A, adv,
                                    lambda Wq: Wq, Wz)
                return lax.fori_loop(0, 4, one_pass, Wy)
            return lax.fori_loop(0, C // 16, sb, Wx)

        W = lax.cond(hi < W + A, fastp, slowp, W)
        return (W, hi)

    def issue(buf_i, buf_v, sem_i, sem_v, chunk):
        off = base + chunk * C
        pltpu.make_async_copy(ind_hbm.at[pl.ds(off, C)],
                              buf_i.at[pl.ds(0, C)], sem_i).start()
        pltpu.make_async_copy(val_hbm.at[pl.ds(off, C)], buf_v, sem_v).start()

    def wait(buf_i, buf_v, sem_i, sem_v, chunk):
        off = base + chunk * C
        pltpu.make_async_copy(ind_hbm.at[pl.ds(off, C)],
                              buf_i.at[pl.ds(0, C)], sem_i).wait()
        pltpu.make_async_copy(val_hbm.at[pl.ds(off, C)], buf_v, sem_v).wait()

    def zw(j, carry):
        win[pl.ds(j * 16, 16)] = z16
        return carry
    lax.fori_loop(0, A // 16, zw, 0)
    ids0[pl.ds(C, 16)] = jnp.zeros((16,), jnp.int32)
    ids1[pl.ds(C, 16)] = jnp.zeros((16,), jnp.int32)
    pltpu.sync_copy(win.at[pl.ds(0, ZSL)], acc.at[pl.ds(s * ZSL, ZSL)])
    plsc.subcore_barrier()

    issue(ids0, vls0, si0, sv0, 0)

    def gbody(g, Wp):
        cA = 2 * g
        wait(ids0, vls0, si0, sv0, cA)
        issue(ids1, vls1, si1, sv1, cA + 1)
        Wp = process(ids0, vls0, Wp)
        wait(ids1, vls1, si1, sv1, cA + 1)
        @pl.when(g < NCH // 2 - 1)
        def _():
            issue(ids0, vls0, si0, sv0, cA + 2)
        Wp = process(ids1, vls1, Wp)
        return Wp

    W, ph = lax.fori_loop(0, NCH // 2, gbody,
                          (jnp.int32(0), jnp.int32(0)))
    flush(W, ph)

    plsc.subcore_barrier()

    @pl.when(s == 0)
    def _():
        pltpu.sync_copy(acc.at[pl.ds(0, OUT_PAD)], out_hbm.at[c])


def _merge_body(a_ref, o_ref):
    o_ref[...] = a_ref[0] + a_ref[1]


def kernel(ind_1, output):
    ind2 = ind_1.reshape(N)
    val2 = output.reshape(N)

    mesh = plsc.VectorSubcoreMesh(core_axis_name="c", subcore_axis_name="s")
    partials = pl.kernel(
        _sc_body,
        out_type=jax.ShapeDtypeStruct((2, OUT_PAD), jnp.float32),
        mesh=mesh,
        compiler_params=pltpu.CompilerParams(needs_layout_passes=False),
        scratch_types=[
            pltpu.VMEM((C + 16,), jnp.int32),
            pltpu.VMEM((C + 16,), jnp.int32),
            pltpu.VMEM((C,), jnp.float32),
            pltpu.VMEM((C,), jnp.float32),
            pltpu.VMEM((A,), jnp.float32),
            pltpu.VMEM((FB,), jnp.int32),
            pltpu.VMEM_SHARED((ACC_PAD,), jnp.float32),
            pltpu.SemaphoreType.DMA,
            pltpu.SemaphoreType.DMA,
            pltpu.SemaphoreType.DMA,
            pltpu.SemaphoreType.DMA,
        ],
    )(ind2, val2)

    merged = pl.pallas_call(
        _merge_body,
        out_shape=jax.ShapeDtypeStruct((OUT_PAD // 128, 128), jnp.float32),
    )(partials.reshape(2, OUT_PAD // 128, 128))
    return merged.reshape(OUT_PAD)[:S]


# speedup vs baseline: 1.0267x; 1.0187x over previous
import jax
import jax.numpy as jnp
from jax import lax
from jax.experimental import pallas as pl
from jax.experimental.pallas import tpu as pltpu
from jax.experimental.pallas import tpu_sc as plsc

N = 6_400_000
S = 100_000
NWORK = 32
PERW = N // NWORK
C = 10_000
NCH = PERW // C
A = 32_768
FB = 4_096
NFB = A // FB
OUT_PAD = 100_352
ACC_PAD = 133_120
ZSL = ACC_PAD // 16


def _sc_body(ind_hbm, val_hbm, out_hbm, ids0, ids1, vls0, vls1, win,
             elemidx, acc, si0, si1, sv0, sv1):
    c = lax.axis_index("c")
    s = lax.axis_index("s")
    w = s * 2 + c
    base = w * PERW
    z16 = jnp.zeros((16,), jnp.float32)
    iota16 = lax.iota(jnp.int32, 16)
    lane15 = iota16 == 15

    def flush(W, ph):
        used = ph - W + 1
        for b in range(NFB):
            @pl.when(b * FB < used)
            def _():
                def fe(j, carry):
                    elemidx[pl.ds(j * 16, 16)] = W + b * FB + j * 16 + iota16
                    return carry
                lax.fori_loop(0, FB // 16, fe, 0)
                pltpu.sync_copy(win.at[pl.ds(b * FB, FB)], acc.at[elemidx],
                                add=True)
                def zz(j, carry):
                    win[pl.ds(b * FB + j * 16, 16)] = z16
                    return carry
                lax.fori_loop(0, FB // 16, zz, 0)

    def align(x):
        return jnp.bitwise_and(x, jnp.int32(-128))

    def scatter(ids_ref, vls_ref, k, W):
        loc = ids_ref[pl.ds(k * 16, 16)] - W
        vv = vls_ref[pl.ds(k * 16, 16)]
        plsc.addupdate_scatter(win, [loc], vv)

    def process(ids_ref, vls_ref, Wp):
        W, ph = Wp
        lo = ids_ref[pl.ds(0, 16)][0]
        hi = ids_ref[pl.ds(C - 16, 16)][15]

        def chunk_flush(a):
            flush(a[0], a[1])
            return align(lo)
        W = lax.cond(hi >= W + A, chunk_flush, lambda a: a[0], (W, ph))

        def fastp(Wx):
            @plsc.parallel_loop(0, C // 16, unroll=4)
            def fb(k):
                b16 = k * 16
                ids = ids_ref[pl.ds(b16, 16)]
                vv = vls_ref[pl.ds(b16, 16)]
                idn = ids_ref[pl.ds(b16 + 1, 16)]
                prefix = plsc.cumsum(vv)
                bnd = ids != idn
                m_end = bnd | lane15
                plsc.addupdate_scatter(win, [ids - Wx], prefix, mask=m_end)
                m_sub = bnd & (~lane15)
                plsc.addupdate_scatter(win, [jnp.clip(idn - Wx, 0, A - 1)],
                                       -prefix, mask=m_sub)
            return Wx

        def slowp(Wx):
            def sb(k, Wy):
                ids = ids_ref[pl.ds(k * 16, 16)]
                vv = vls_ref[pl.ds(k * 16, 16)]
                def vec_flush(Wz):
                    flush(Wz, Wz + A - 1)
                    return align(ids[0])
                Wy = lax.cond(ids[15] >= Wy + A, vec_flush, lambda Wz: Wz, Wy)
                def one_pass(p, Wz):
                    loc = ids - Wz
                    m = (loc >= 0) & (loc < A)
                    plsc.addupdate_scatter(
                        win, [jnp.clip(loc, 0, A - 1)], vv, mask=m)
                    def adv(Wq):
                        flush(Wq, Wq + A - 1)
                        return Wq + A
                    return lax.cond(ids[15] >= Wz + A, adv,
                                    lambda Wq: Wq, Wz)
                return lax.fori_loop(0, 4, one_pass, Wy)
            return lax.fori_loop(0, C // 16, sb, Wx)

        W = lax.cond(hi < W + A, fastp, slowp, W)
        return (W, hi)

    def issue(buf_i, buf_v, sem_i, sem_v, chunk):
        off = base + chunk * C
        pltpu.make_async_copy(ind_hbm.at[pl.ds(off, C)],
                              buf_i.at[pl.ds(0, C)], sem_i).start()
        pltpu.make_async_copy(val_hbm.at[pl.ds(off, C)], buf_v, sem_v).start()

    def wait(buf_i, buf_v, sem_i, sem_v, chunk):
        off = base + chunk * C
        pltpu.make_async_copy(ind_hbm.at[pl.ds(off, C)],
                              buf_i.at[pl.ds(0, C)], sem_i).wait()
        pltpu.make_async_copy(val_hbm.at[pl.ds(off, C)], buf_v, sem_v).wait()

    def zw(j, carry):
        win[pl.ds(j * 16, 16)] = z16
        return carry
    lax.fori_loop(0, A // 16, zw, 0)
    ids0[pl.ds(C, 16)] = jnp.zeros((16,), jnp.int32)
    ids1[pl.ds(C, 16)] = jnp.zeros((16,), jnp.int32)
    pltpu.sync_copy(win.at[pl.ds(0, ZSL)], acc.at[pl.ds(s * ZSL, ZSL)])
    plsc.subcore_barrier()

    issue(ids0, vls0, si0, sv0, 0)

    def gbody(g, Wp):
        cA = 2 * g
        wait(ids0, vls0, si0, sv0, cA)
        issue(ids1, vls1, si1, sv1, cA + 1)
        Wp = process(ids0, vls0, Wp)
        wait(ids1, vls1, si1, sv1, cA + 1)
        @pl.when(g < NCH // 2 - 1)
        def _():
            issue(ids0, vls0, si0, sv0, cA + 2)
        Wp = process(ids1, vls1, Wp)
        return Wp

    W, ph = lax.fori_loop(0, NCH // 2, gbody,
                          (jnp.int32(0), jnp.int32(0)))
    flush(W, ph)

    plsc.subcore_barrier()

    @pl.when(s == 0)
    def _():
        pltpu.sync_copy(acc.at[pl.ds(0, OUT_PAD)], out_hbm.at[c])


def _merge_body(a_ref, o_ref):
    o_ref[...] = a_ref[0] + a_ref[1]


def kernel(ind_1, output):
    ind2 = ind_1.reshape(N)
    val2 = output.reshape(N)

    mesh = plsc.VectorSubcoreMesh(core_axis_name="c", subcore_axis_name="s")
    partials = pl.kernel(
        _sc_body,
        out_type=jax.ShapeDtypeStruct((2, OUT_PAD), jnp.float32),
        mesh=mesh,
        compiler_params=pltpu.CompilerParams(needs_layout_passes=False,
                                             use_tc_tiling_on_sc=False),
        scratch_types=[
            pltpu.VMEM((C + 16,), jnp.int32),
            pltpu.VMEM((C + 16,), jnp.int32),
            pltpu.VMEM((C,), jnp.float32),
            pltpu.VMEM((C,), jnp.float32),
            pltpu.VMEM((A,), jnp.float32),
            pltpu.VMEM((FB,), jnp.int32),
            pltpu.VMEM_SHARED((ACC_PAD,), jnp.float32),
            pltpu.SemaphoreType.DMA,
            pltpu.SemaphoreType.DMA,
            pltpu.SemaphoreType.DMA,
            pltpu.SemaphoreType.DMA,
        ],
    )(ind2, val2)

    merged = pl.pallas_call(
        _merge_body,
        out_shape=jax.ShapeDtypeStruct((OUT_PAD // 128, 128), jnp.float32),
    )(partials.reshape(2, OUT_PAD // 128, 128))
    return merged.reshape(OUT_PAD)[:S]
